# baseline (device time: 25471 ns/iter reference)
import jax
import jax.numpy as jnp
from jax import lax
from jax.experimental import pallas as pl
from jax.experimental.pallas import tpu as pltpu

NY, NZ = 4, 4
NYZ = NY * NZ


def kernel(Q, K, V):
    B, SKV, H, D = K.shape
    HD = H * D
    scale = D ** -0.5
    assert B == NYZ

    my_b = lax.axis_index("y") * NZ + lax.axis_index("z")
    bf16 = jnp.bfloat16
    qb = lax.dynamic_slice_in_dim(Q, my_b, 1, 0).reshape(HD, 1)
    k2 = lax.dynamic_slice_in_dim(K, my_b, 1, 0).reshape(SKV, HD).astype(bf16)
    v2 = lax.dynamic_slice_in_dim(V, my_b, 1, 0).reshape(SKV, HD)

    def body(q_ref, k_ref, v_ref, o_ref,
             obuf, accb, mb, lb, racc, rm, rl,
             xsend, xrecv, bss, brs):
        my_x = lax.axis_index("x")
        my_y = lax.axis_index("y")
        my_z = lax.axis_index("z")
        my_yz = my_y * NZ + my_z
        peer_x = (1 - my_x, my_y, my_z)

        bsem = pltpu.get_barrier_semaphore()
        pl.semaphore_signal(
            bsem, inc=1, device_id=peer_x,
            device_id_type=pl.DeviceIdType.MESH,
        )
        for dy in range(NY):
            for dz in range(NZ):
                dyz = dy * NZ + dz

                @pl.when(dyz != my_yz)
                def _():
                    pl.semaphore_signal(
                        bsem, inc=1, device_id=(my_x, dy, dz),
                        device_id_type=pl.DeviceIdType.MESH,
                    )
        pl.semaphore_wait(bsem, NYZ)

        row_h = lax.broadcasted_iota(jnp.int32, (HD, H), 0) // D
        col_h = lax.broadcasted_iota(jnp.int32, (HD, H), 1)
        qmask = row_h == col_h
        prow = lax.broadcasted_iota(jnp.int32, (H, HD), 0)
        pcol = lax.broadcasted_iota(jnp.int32, (H, HD), 1) // D
        pmask = prow == pcol

        qf = q_ref[...]
        qd = jnp.where(
            qmask, jnp.broadcast_to(qf, (HD, H)), 0.0
        ).astype(jnp.bfloat16)

        sm = lax.dot_general(
            k_ref[...], qd, (((1,), (0,)), ((), ())),
            preferred_element_type=jnp.float32,
        ) * scale
        m = jnp.max(sm, axis=0, keepdims=True)
        p = jnp.exp(sm - m)
        l = jnp.sum(p, axis=0, keepdims=True)

        emat = jnp.where(pmask, 1.0, 0.0)
        pe = lax.dot_general(
            p, emat, (((1,), (0,)), ((), ())),
            preferred_element_type=jnp.float32,
        )
        acc = jnp.sum(pe * v_ref[...], axis=0, keepdims=True)

        accb[...] = acc
        mb[...] = m
        lb[...] = l

        rdmas = []
        for i, (src, dst) in enumerate([(accb, racc), (mb, rm), (lb, rl)]):
            rdma = pltpu.make_async_remote_copy(
                src_ref=src,
                dst_ref=dst,
                send_sem=xsend.at[i],
                recv_sem=xrecv.at[i],
                device_id=peer_x,
                device_id_type=pl.DeviceIdType.MESH,
            )
            rdma.start()
            rdmas.append(rdma)
        for rdma in rdmas:
            rdma.wait()

        m_r = rm[...]
        l_r = rl[...]
        mn = jnp.maximum(m, m_r)
        ea = jnp.exp(m - mn)
        eb = jnp.exp(m_r - mn)
        ln = l * ea + l_r * eb
        dn = (((1,), (0,)), ((), ()))
        eae = lax.dot_general(ea, emat, dn,
                              preferred_element_type=jnp.float32)
        ebe = lax.dot_general(eb, emat, dn,
                              preferred_element_type=jnp.float32)
        lne = lax.dot_general(ln, emat, dn,
                              preferred_element_type=jnp.float32)
        obuf[my_yz] = (acc * eae + racc[...] * ebe) / lne

        for dy in range(NY):
            for dz in range(NZ):
                dyz = dy * NZ + dz

                @pl.when(dyz != my_yz)
                def _():
                    rdma = pltpu.make_async_remote_copy(
                        src_ref=obuf.at[my_yz],
                        dst_ref=obuf.at[my_yz],
                        send_sem=bss.at[dyz],
                        recv_sem=brs.at[my_yz],
                        device_id=(my_x, dy, dz),
                        device_id_type=pl.DeviceIdType.MESH,
                    )
                    rdma.start()

        for j in range(NYZ):

            @pl.when(j != my_yz)
            def _():
                rcv = pltpu.make_async_remote_copy(
                    src_ref=obuf.at[j],
                    dst_ref=obuf.at[j],
                    send_sem=bss.at[j],
                    recv_sem=brs.at[j],
                    device_id=peer_x,
                    device_id_type=pl.DeviceIdType.MESH,
                )
                rcv.wait_recv()
                snd = pltpu.make_async_remote_copy(
                    src_ref=obuf.at[my_yz],
                    dst_ref=obuf.at[j],
                    send_sem=bss.at[j],
                    recv_sem=brs.at[j],
                    device_id=peer_x,
                    device_id_type=pl.DeviceIdType.MESH,
                )
                snd.wait_send()

        o_ref[...] = obuf[...]

    out = pl.pallas_call(
        body,
        in_specs=[
            pl.BlockSpec(memory_space=pltpu.VMEM),
            pl.BlockSpec(memory_space=pltpu.VMEM),
            pl.BlockSpec(memory_space=pltpu.VMEM),
        ],
        out_specs=pl.BlockSpec(memory_space=pltpu.VMEM),
        out_shape=jax.ShapeDtypeStruct((B, 1, HD), jnp.float32),
        scratch_shapes=[
            pltpu.VMEM((B, 1, HD), jnp.float32),
            pltpu.VMEM((1, HD), jnp.float32),
            pltpu.VMEM((1, H), jnp.float32),
            pltpu.VMEM((1, H), jnp.float32),
            pltpu.VMEM((1, HD), jnp.float32),
            pltpu.VMEM((1, H), jnp.float32),
            pltpu.VMEM((1, H), jnp.float32),
            pltpu.SemaphoreType.DMA((3,)),
            pltpu.SemaphoreType.DMA((3,)),
            pltpu.SemaphoreType.DMA((NYZ,)),
            pltpu.SemaphoreType.DMA((NYZ,)),
        ],
        compiler_params=pltpu.CompilerParams(collective_id=0),
    )(qb, k2, v2)
    return out.reshape(B, 1, H, D)


# device time: 22321 ns/iter; 1.1411x vs baseline; 1.1411x over previous
import jax
import jax.numpy as jnp
from jax import lax
from jax.experimental import pallas as pl
from jax.experimental.pallas import tpu as pltpu

NY, NZ = 4, 4
NYZ = NY * NZ


def kernel(Q, K, V):
    B, SKV, H, D = K.shape
    HD = H * D
    scale = D ** -0.5
    assert B == NYZ

    my_b = lax.axis_index("y") * NZ + lax.axis_index("z")
    bf16 = jnp.bfloat16
    qb = lax.dynamic_slice_in_dim(Q, my_b, 1, 0).reshape(HD, 1)
    k2 = lax.dynamic_slice_in_dim(K, my_b, 1, 0).reshape(SKV, HD).astype(bf16)
    v2 = lax.dynamic_slice_in_dim(V, my_b, 1, 0).reshape(SKV, HD).astype(bf16)

    def body(q_ref, k_ref, v_ref, o_ref,
             obuf, accb, mb, lb, racc, rm, rl,
             xsend, xrecv, bss, brs):
        my_x = lax.axis_index("x")
        my_y = lax.axis_index("y")
        my_z = lax.axis_index("z")
        my_yz = my_y * NZ + my_z
        peer_x = (1 - my_x, my_y, my_z)

        bsem = pltpu.get_barrier_semaphore()
        pl.semaphore_signal(
            bsem, inc=1, device_id=peer_x,
            device_id_type=pl.DeviceIdType.MESH,
        )
        for dy in range(NY):
            for dz in range(NZ):
                dyz = dy * NZ + dz

                @pl.when(dyz != my_yz)
                def _():
                    pl.semaphore_signal(
                        bsem, inc=1, device_id=(my_x, dy, dz),
                        device_id_type=pl.DeviceIdType.MESH,
                    )
        pl.semaphore_wait(bsem, NYZ)

        row_h = lax.broadcasted_iota(jnp.int32, (HD, H), 0) // D
        col_h = lax.broadcasted_iota(jnp.int32, (HD, H), 1)
        qmask = row_h == col_h
        prow = lax.broadcasted_iota(jnp.int32, (H, HD), 0)
        pcol = lax.broadcasted_iota(jnp.int32, (H, HD), 1) // D
        pmask = prow == pcol

        qf = q_ref[...]
        qd = jnp.where(
            qmask, jnp.broadcast_to(qf, (HD, H)), 0.0
        ).astype(jnp.bfloat16)

        sm = lax.dot_general(
            k_ref[...], qd, (((1,), (0,)), ((), ())),
            preferred_element_type=jnp.float32,
        ) * scale
        m = jnp.max(sm, axis=0, keepdims=True)
        p = jnp.exp(sm - m)
        l = jnp.sum(p, axis=0, keepdims=True)

        ptv = lax.dot_general(
            p.astype(jnp.bfloat16), v_ref[...], (((0,), (0,)), ((), ())),
            preferred_element_type=jnp.float32,
        )
        acc = jnp.sum(
            jnp.where(pmask, ptv, 0.0), axis=0, keepdims=True
        )

        accb[...] = acc
        mb[...] = m
        lb[...] = l

        rdmas = []
        for i, (src, dst) in enumerate([(accb, racc), (mb, rm), (lb, rl)]):
            rdma = pltpu.make_async_remote_copy(
                src_ref=src,
                dst_ref=dst,
                send_sem=xsend.at[i],
                recv_sem=xrecv.at[i],
                device_id=peer_x,
                device_id_type=pl.DeviceIdType.MESH,
            )
            rdma.start()
            rdmas.append(rdma)
        for rdma in rdmas:
            rdma.wait()

        m_r = rm[...]
        l_r = rl[...]
        mn = jnp.maximum(m, m_r)
        ea = jnp.exp(m - mn)
        eb = jnp.exp(m_r - mn)
        ln = l * ea + l_r * eb
        emat = jnp.where(pmask, 1.0, 0.0)
        dn = (((1,), (0,)), ((), ()))
        eae = lax.dot_general(ea, emat, dn,
                              preferred_element_type=jnp.float32)
        ebe = lax.dot_general(eb, emat, dn,
                              preferred_element_type=jnp.float32)
        lne = lax.dot_general(ln, emat, dn,
                              preferred_element_type=jnp.float32)
        obuf[my_yz] = (acc * eae + racc[...] * ebe) / lne

        for dy in range(NY):
            for dz in range(NZ):
                dyz = dy * NZ + dz

                @pl.when(dyz != my_yz)
                def _():
                    rdma = pltpu.make_async_remote_copy(
                        src_ref=obuf.at[my_yz],
                        dst_ref=obuf.at[my_yz],
                        send_sem=bss.at[dyz],
                        recv_sem=brs.at[my_yz],
                        device_id=(my_x, dy, dz),
                        device_id_type=pl.DeviceIdType.MESH,
                    )
                    rdma.start()

        for j in range(NYZ):

            @pl.when(j != my_yz)
            def _():
                rcv = pltpu.make_async_remote_copy(
                    src_ref=obuf.at[j],
                    dst_ref=obuf.at[j],
                    send_sem=bss.at[j],
                    recv_sem=brs.at[j],
                    device_id=peer_x,
                    device_id_type=pl.DeviceIdType.MESH,
                )
                rcv.wait_recv()
                snd = pltpu.make_async_remote_copy(
                    src_ref=obuf.at[my_yz],
                    dst_ref=obuf.at[j],
                    send_sem=bss.at[j],
                    recv_sem=brs.at[j],
                    device_id=peer_x,
                    device_id_type=pl.DeviceIdType.MESH,
                )
                snd.wait_send()

        o_ref[...] = obuf[...]

    out = pl.pallas_call(
        body,
        in_specs=[
            pl.BlockSpec(memory_space=pltpu.VMEM),
            pl.BlockSpec(memory_space=pltpu.VMEM),
            pl.BlockSpec(memory_space=pltpu.VMEM),
        ],
        out_specs=pl.BlockSpec(memory_space=pltpu.VMEM),
        out_shape=jax.ShapeDtypeStruct((B, 1, HD), jnp.float32),
        scratch_shapes=[
            pltpu.VMEM((B, 1, HD), jnp.float32),
            pltpu.VMEM((1, HD), jnp.float32),
            pltpu.VMEM((1, H), jnp.float32),
            pltpu.VMEM((1, H), jnp.float32),
            pltpu.VMEM((1, HD), jnp.float32),
            pltpu.VMEM((1, H), jnp.float32),
            pltpu.VMEM((1, H), jnp.float32),
            pltpu.SemaphoreType.DMA((3,)),
            pltpu.SemaphoreType.DMA((3,)),
            pltpu.SemaphoreType.DMA((NYZ,)),
            pltpu.SemaphoreType.DMA((NYZ,)),
        ],
        compiler_params=pltpu.CompilerParams(collective_id=0),
    )(qb, k2, v2)
    return out.reshape(B, 1, H, D)


# device time: 22305 ns/iter; 1.1419x vs baseline; 1.0007x over previous
import jax
import jax.numpy as jnp
from jax import lax
from jax.experimental import pallas as pl
from jax.experimental.pallas import tpu as pltpu

NY, NZ = 4, 4
NYZ = NY * NZ


def kernel(Q, K, V):
    B, SKV, H, D = K.shape
    HD = H * D
    scale = D ** -0.5
    assert B == NYZ

    my_b = lax.axis_index("y") * NZ + lax.axis_index("z")
    bf16 = jnp.bfloat16
    qb = lax.dynamic_slice_in_dim(Q, my_b, 1, 0).reshape(HD, 1)
    k2 = lax.dynamic_slice_in_dim(K, my_b, 1, 0).reshape(SKV, HD).astype(bf16)
    v2 = lax.dynamic_slice_in_dim(V, my_b, 1, 0).reshape(SKV, HD).astype(bf16)

    def body(q_ref, k_ref, v_ref, o_ref,
             obuf, accb, mb, lb, racc, rm, rl,
             xsend, xrecv, bss, brs):
        my_x = lax.axis_index("x")
        my_y = lax.axis_index("y")
        my_z = lax.axis_index("z")
        my_yz = my_y * NZ + my_z
        peer_x = (1 - my_x, my_y, my_z)

        bsem = pltpu.get_barrier_semaphore()
        pl.semaphore_signal(
            bsem, inc=1, device_id=peer_x,
            device_id_type=pl.DeviceIdType.MESH,
        )
        for dy in range(NY):
            for dz in range(NZ):
                dyz = dy * NZ + dz

                @pl.when(dyz != my_yz)
                def _():
                    pl.semaphore_signal(
                        bsem, inc=1, device_id=(my_x, dy, dz),
                        device_id_type=pl.DeviceIdType.MESH,
                    )
        pl.semaphore_wait(bsem, NYZ)

        row_h = lax.broadcasted_iota(jnp.int32, (HD, H), 0) // D
        col_h = lax.broadcasted_iota(jnp.int32, (HD, H), 1)
        qmask = row_h == col_h
        prow = lax.broadcasted_iota(jnp.int32, (H, HD), 0)
        pcol = lax.broadcasted_iota(jnp.int32, (H, HD), 1) // D
        pmask = prow == pcol

        qf = q_ref[...]
        qd = jnp.where(
            qmask, jnp.broadcast_to(qf, (HD, H)), 0.0
        ).astype(jnp.bfloat16)

        sm = lax.dot_general(
            k_ref[...], qd, (((1,), (0,)), ((), ())),
            preferred_element_type=jnp.float32,
        ) * scale
        m = jnp.max(sm, axis=0, keepdims=True)
        p = jnp.exp(sm - m)
        l = jnp.sum(p, axis=0, keepdims=True)

        mb[...] = m
        lb[...] = l
        stat_rdmas = []
        for i, (src, dst) in enumerate([(mb, rm), (lb, rl)]):
            rdma = pltpu.make_async_remote_copy(
                src_ref=src,
                dst_ref=dst,
                send_sem=xsend.at[i],
                recv_sem=xrecv.at[i],
                device_id=peer_x,
                device_id_type=pl.DeviceIdType.MESH,
            )
            rdma.start()
            stat_rdmas.append(rdma)

        ptv = lax.dot_general(
            p.astype(jnp.bfloat16), v_ref[...], (((0,), (0,)), ((), ())),
            preferred_element_type=jnp.float32,
        )
        acc = jnp.sum(
            jnp.where(pmask, ptv, 0.0), axis=0, keepdims=True
        )
        accb[...] = acc
        acc_rdma = pltpu.make_async_remote_copy(
            src_ref=accb,
            dst_ref=racc,
            send_sem=xsend.at[2],
            recv_sem=xrecv.at[2],
            device_id=peer_x,
            device_id_type=pl.DeviceIdType.MESH,
        )
        acc_rdma.start()

        for rdma in stat_rdmas:
            rdma.wait()
        m_r = rm[...]
        l_r = rl[...]
        mn = jnp.maximum(m, m_r)
        ea = jnp.exp(m - mn)
        eb = jnp.exp(m_r - mn)
        ln = l * ea + l_r * eb
        emat = jnp.where(pmask, 1.0, 0.0)
        dn = (((1,), (0,)), ((), ()))
        eae = lax.dot_general(ea, emat, dn,
                              preferred_element_type=jnp.float32)
        ebe = lax.dot_general(eb, emat, dn,
                              preferred_element_type=jnp.float32)
        lne = lax.dot_general(ln, emat, dn,
                              preferred_element_type=jnp.float32)
        acc_rdma.wait()
        obuf[my_yz] = (acc * eae + racc[...] * ebe) / lne

        for dy in range(NY):
            for dz in range(NZ):
                dyz = dy * NZ + dz

                @pl.when(dyz != my_yz)
                def _():
                    rdma = pltpu.make_async_remote_copy(
                        src_ref=obuf.at[my_yz],
                        dst_ref=obuf.at[my_yz],
                        send_sem=bss.at[dyz],
                        recv_sem=brs.at[my_yz],
                        device_id=(my_x, dy, dz),
                        device_id_type=pl.DeviceIdType.MESH,
                    )
                    rdma.start()

        for j in range(NYZ):

            @pl.when(j != my_yz)
            def _():
                rcv = pltpu.make_async_remote_copy(
                    src_ref=obuf.at[j],
                    dst_ref=obuf.at[j],
                    send_sem=bss.at[j],
                    recv_sem=brs.at[j],
                    device_id=peer_x,
                    device_id_type=pl.DeviceIdType.MESH,
                )
                rcv.wait_recv()
                snd = pltpu.make_async_remote_copy(
                    src_ref=obuf.at[my_yz],
                    dst_ref=obuf.at[j],
                    send_sem=bss.at[j],
                    recv_sem=brs.at[j],
                    device_id=peer_x,
                    device_id_type=pl.DeviceIdType.MESH,
                )
                snd.wait_send()

        o_ref[...] = obuf[...]

    out = pl.pallas_call(
        body,
        in_specs=[
            pl.BlockSpec(memory_space=pltpu.VMEM),
            pl.BlockSpec(memory_space=pltpu.VMEM),
            pl.BlockSpec(memory_space=pltpu.VMEM),
        ],
        out_specs=pl.BlockSpec(memory_space=pltpu.VMEM),
        out_shape=jax.ShapeDtypeStruct((B, 1, HD), jnp.float32),
        scratch_shapes=[
            pltpu.VMEM((B, 1, HD), jnp.float32),
            pltpu.VMEM((1, HD), jnp.float32),
            pltpu.VMEM((1, H), jnp.float32),
            pltpu.VMEM((1, H), jnp.float32),
            pltpu.VMEM((1, HD), jnp.float32),
            pltpu.VMEM((1, H), jnp.float32),
            pltpu.VMEM((1, H), jnp.float32),
            pltpu.SemaphoreType.DMA((3,)),
            pltpu.SemaphoreType.DMA((3,)),
            pltpu.SemaphoreType.DMA((NYZ,)),
            pltpu.SemaphoreType.DMA((NYZ,)),
        ],
        compiler_params=pltpu.CompilerParams(collective_id=0),
    )(qb, k2, v2)
    return out.reshape(B, 1, H, D)
